# baseline (device time: 789242 ns/iter reference)
import jax
import jax.numpy as jnp
from jax import lax
from jax.experimental import pallas as pl
from jax.experimental.pallas import tpu as pltpu

N_DEV = 32


def kernel(x, w_mat, scale_x, scale_w):
    m_tot, k_loc = x.shape
    _, n = w_mat.shape
    m_per = m_tot // N_DEV

    def body(x_ref, w_ref, sx_ref, sw_ref, out_ref,
             w_bf, send_buf, recv_buf, send_sems, recv_sems, credit_sem):
        my = lax.axis_index("i")
        left = lax.rem(my - 1 + N_DEV, N_DEV)
        right = lax.rem(my + 1, N_DEV)

        barrier = pltpu.get_barrier_semaphore()
        for nbr in (left, right):
            pl.semaphore_signal(
                barrier, inc=1,
                device_id=(nbr,), device_id_type=pl.DeviceIdType.MESH,
            )
        pl.semaphore_wait(barrier, 2)

        w_bf[:, :] = w_ref[:, :].astype(jnp.bfloat16)

        def chunk_partial(c):
            rows = pl.ds(c * m_per, m_per)
            return jnp.dot(
                x_ref[rows, :].astype(jnp.bfloat16), w_bf[:, :],
                preferred_element_type=jnp.float32,
            )

        for s in range(N_DEV - 1):
            c = lax.rem(my - 1 - s + 2 * N_DEV, N_DEV)
            part = chunk_partial(c)
            if s > 0:
                part = part + recv_buf[(s - 1) % 2].astype(jnp.float32)
            send_buf[s % 2] = part.astype(jnp.bfloat16)
            if 1 <= s <= N_DEV - 3:
                pl.semaphore_signal(
                    credit_sem, inc=1,
                    device_id=(left,), device_id_type=pl.DeviceIdType.MESH,
                )
            if s >= 2:
                pl.semaphore_wait(credit_sem, 1)
            rdma = pltpu.make_async_remote_copy(
                src_ref=send_buf.at[s % 2],
                dst_ref=recv_buf.at[s % 2],
                send_sem=send_sems.at[s % 2],
                recv_sem=recv_sems.at[s % 2],
                device_id=(right,),
                device_id_type=pl.DeviceIdType.MESH,
            )
            rdma.start()
            rdma.wait()

        acc = chunk_partial(my) + recv_buf[(N_DEV - 2) % 2].astype(jnp.float32)
        y = acc * (sx_ref[0] * sw_ref[0])
        out_ref[:, :] = y * jax.nn.sigmoid(y)

    return pl.pallas_call(
        body,
        out_shape=jax.ShapeDtypeStruct((m_per, n), jnp.float32),
        in_specs=[
            pl.BlockSpec(memory_space=pltpu.VMEM),
            pl.BlockSpec(memory_space=pltpu.VMEM),
            pl.BlockSpec(memory_space=pltpu.SMEM),
            pl.BlockSpec(memory_space=pltpu.SMEM),
        ],
        out_specs=pl.BlockSpec(memory_space=pltpu.VMEM),
        scratch_shapes=[
            pltpu.VMEM((k_loc, n), jnp.bfloat16),
            pltpu.VMEM((2, m_per, n), jnp.bfloat16),
            pltpu.VMEM((2, m_per, n), jnp.bfloat16),
            pltpu.SemaphoreType.DMA((2,)),
            pltpu.SemaphoreType.DMA((2,)),
            pltpu.SemaphoreType.REGULAR,
        ],
        compiler_params=pltpu.CompilerParams(collective_id=0),
    )(x, w_mat, scale_x, scale_w)


# device time: 450454 ns/iter; 1.7521x vs baseline; 1.7521x over previous
import jax
import jax.numpy as jnp
from jax import lax
from jax.experimental import pallas as pl
from jax.experimental.pallas import tpu as pltpu

N_DEV = 32

PERM = [0, 8, 16, 24, 27, 19, 11, 12, 20, 28, 31, 23, 15, 7, 4, 3,
        2, 5, 6, 14, 22, 30, 29, 21, 13, 10, 18, 26, 25, 17, 9, 1]


def kernel(x, w_mat, scale_x, scale_w):
    m_tot, k_loc = x.shape
    _, n = w_mat.shape
    m_per = m_tot // N_DEV
    n2 = n // 2

    my = lax.axis_index("i")
    perm = jnp.array(PERM, dtype=jnp.int32)
    rpos = jnp.argmax(perm == my).astype(jnp.int32)
    succ = perm[(rpos + 1) % N_DEV]
    pred = perm[(rpos - 1) % N_DEV]
    s_arr = jnp.arange(N_DEV, dtype=jnp.int32)
    chunks_f = perm[(rpos - 1 - s_arr) % N_DEV]
    chunks_b = perm[(rpos + 1 + s_arr) % N_DEV]
    meta = jnp.concatenate([jnp.stack([succ, pred]), chunks_f, chunks_b])

    def body(x_ref, w_ref, sx_ref, sw_ref, meta_ref, out_ref,
             w_bf, sbF, rbF, sbB, rbB, ssF, rsF, ssB, rsB, credF, credB):
        nxt = meta_ref[0]
        prv = meta_ref[1]

        barrier = pltpu.get_barrier_semaphore()
        for nbr in (nxt, prv):
            pl.semaphore_signal(
                barrier, inc=1,
                device_id=(nbr,), device_id_type=pl.DeviceIdType.MESH,
            )
        pl.semaphore_wait(barrier, 2)

        w_bf[:, :] = w_ref[:, :].astype(jnp.bfloat16)

        def part_f(c):
            rows = pl.ds(c * m_per, m_per)
            return jnp.dot(x_ref[rows, :].astype(jnp.bfloat16),
                           w_bf[:, :n2], preferred_element_type=jnp.float32)

        def part_b(c):
            rows = pl.ds(c * m_per, m_per)
            return jnp.dot(x_ref[rows, :].astype(jnp.bfloat16),
                           w_bf[:, n2:], preferred_element_type=jnp.float32)

        def mk(sb, rb, ss, rs, slot, dev):
            return pltpu.make_async_remote_copy(
                src_ref=sb.at[slot], dst_ref=rb.at[slot],
                send_sem=ss.at[slot], recv_sem=rs.at[slot],
                device_id=(dev,), device_id_type=pl.DeviceIdType.MESH,
            )

        rdF = {}
        rdB = {}

        sbF[0] = part_f(meta_ref[2]).astype(jnp.bfloat16)
        sbB[0] = part_b(meta_ref[2 + N_DEV]).astype(jnp.bfloat16)
        rdF[0] = mk(sbF, rbF, ssF, rsF, 0, nxt)
        rdB[0] = mk(sbB, rbB, ssB, rsB, 0, prv)
        rdF[0].start()
        rdB[0].start()

        for s in range(1, N_DEV - 1):
            slot = s % 2
            pF = part_f(meta_ref[2 + s])
            pB = part_b(meta_ref[2 + N_DEV + s])
            rdF[s - 1].wait_recv()
            rdB[s - 1].wait_recv()
            vF = (pF + rbF[1 - slot].astype(jnp.float32)).astype(jnp.bfloat16)
            vB = (pB + rbB[1 - slot].astype(jnp.float32)).astype(jnp.bfloat16)
            if s >= 2:
                rdF[s - 2].wait_send()
                rdB[s - 2].wait_send()
            sbF[slot] = vF
            sbB[slot] = vB
            if s <= N_DEV - 3:
                pl.semaphore_signal(credF, inc=1, device_id=(prv,),
                                    device_id_type=pl.DeviceIdType.MESH)
                pl.semaphore_signal(credB, inc=1, device_id=(nxt,),
                                    device_id_type=pl.DeviceIdType.MESH)
            if s >= 2:
                pl.semaphore_wait(credF, 1)
                pl.semaphore_wait(credB, 1)
            rdF[s] = mk(sbF, rbF, ssF, rsF, slot, nxt)
            rdB[s] = mk(sbB, rbB, ssB, rsB, slot, prv)
            rdF[s].start()
            rdB[s].start()

        me = lax.axis_index("i")
        pF = part_f(me)
        pB = part_b(me)
        rdF[N_DEV - 2].wait_recv()
        rdB[N_DEV - 2].wait_recv()
        last = (N_DEV - 2) % 2
        accF = pF + rbF[last].astype(jnp.float32)
        accB = pB + rbB[last].astype(jnp.float32)
        scale = sx_ref[0] * sw_ref[0]
        yF = accF * scale
        yB = accB * scale
        out_ref[:, :n2] = yF * jax.nn.sigmoid(yF)
        out_ref[:, n2:] = yB * jax.nn.sigmoid(yB)
        for s in (N_DEV - 3, N_DEV - 2):
            rdF[s].wait_send()
            rdB[s].wait_send()

    return pl.pallas_call(
        body,
        out_shape=jax.ShapeDtypeStruct((m_per, n), jnp.float32),
        in_specs=[
            pl.BlockSpec(memory_space=pltpu.VMEM),
            pl.BlockSpec(memory_space=pltpu.VMEM),
            pl.BlockSpec(memory_space=pltpu.SMEM),
            pl.BlockSpec(memory_space=pltpu.SMEM),
            pl.BlockSpec(memory_space=pltpu.SMEM),
        ],
        out_specs=pl.BlockSpec(memory_space=pltpu.VMEM),
        scratch_shapes=[
            pltpu.VMEM((k_loc, n), jnp.bfloat16),
            pltpu.VMEM((2, m_per, n2), jnp.bfloat16),
            pltpu.VMEM((2, m_per, n2), jnp.bfloat16),
            pltpu.VMEM((2, m_per, n2), jnp.bfloat16),
            pltpu.VMEM((2, m_per, n2), jnp.bfloat16),
            pltpu.SemaphoreType.DMA((2,)),
            pltpu.SemaphoreType.DMA((2,)),
            pltpu.SemaphoreType.DMA((2,)),
            pltpu.SemaphoreType.DMA((2,)),
            pltpu.SemaphoreType.REGULAR,
            pltpu.SemaphoreType.REGULAR,
        ],
        compiler_params=pltpu.CompilerParams(collective_id=0),
    )(x, w_mat, scale_x, scale_w, meta)


# device time: 367630 ns/iter; 2.1468x vs baseline; 1.2253x over previous
import jax
import jax.numpy as jnp
from jax import lax
from jax.experimental import pallas as pl
from jax.experimental.pallas import tpu as pltpu

N_DEV = 32
N_PIPE = 4

PERM = [0, 8, 16, 24, 27, 19, 11, 12, 20, 28, 31, 23, 15, 7, 4, 3,
        2, 5, 6, 14, 22, 30, 29, 21, 13, 10, 18, 26, 25, 17, 9, 1]


def kernel(x, w_mat, scale_x, scale_w):
    m_tot, k_loc = x.shape
    _, n = w_mat.shape
    m_per = m_tot // N_DEV
    nq = n // N_PIPE

    my = lax.axis_index("i")
    perm = jnp.array(PERM, dtype=jnp.int32)
    rpos = jnp.argmax(perm == my).astype(jnp.int32)
    succ = perm[(rpos + 1) % N_DEV]
    pred = perm[(rpos - 1) % N_DEV]
    s_arr = jnp.arange(N_DEV, dtype=jnp.int32)
    chunks_f = perm[(rpos - 1 - s_arr) % N_DEV]
    chunks_b = perm[(rpos + 1 + s_arr) % N_DEV]
    meta = jnp.concatenate([jnp.stack([succ, pred]), chunks_f, chunks_b])

    def body(x_ref, w_ref, sx_ref, sw_ref, meta_ref, out_ref, w_bf, *scr):
        sbs = scr[0:N_PIPE]
        rbs = scr[N_PIPE:2 * N_PIPE]
        sss = scr[2 * N_PIPE:3 * N_PIPE]
        rss = scr[3 * N_PIPE:4 * N_PIPE]
        creds = scr[4 * N_PIPE:5 * N_PIPE]

        nxt = meta_ref[0]
        prv = meta_ref[1]

        barrier = pltpu.get_barrier_semaphore()
        for nbr in (nxt, prv):
            pl.semaphore_signal(
                barrier, inc=1,
                device_id=(nbr,), device_id_type=pl.DeviceIdType.MESH,
            )
        pl.semaphore_wait(barrier, 2)

        w_bf[:, :] = w_ref[:, :].astype(jnp.bfloat16)

        cols = [0, nq, 2 * nq, 3 * nq]
        is_fwd = [True, True, False, False]
        order = [0, 2, 1, 3]

        def chunk_at(p, s):
            return meta_ref[2 + (0 if is_fwd[p] else N_DEV) + s]

        def dst(p):
            return nxt if is_fwd[p] else prv

        def src(p):
            return prv if is_fwd[p] else nxt

        def part(p, c):
            rows = pl.ds(c * m_per, m_per)
            return jnp.dot(
                x_ref[rows, :].astype(jnp.bfloat16),
                w_bf[:, cols[p]:cols[p] + nq],
                preferred_element_type=jnp.float32,
            )

        def mk(p, slot):
            return pltpu.make_async_remote_copy(
                src_ref=sbs[p].at[slot], dst_ref=rbs[p].at[slot],
                send_sem=sss[p].at[slot], recv_sem=rss[p].at[slot],
                device_id=(dst(p),), device_id_type=pl.DeviceIdType.MESH,
            )

        rd = {p: {} for p in range(N_PIPE)}

        for p in order:
            sbs[p][0] = part(p, chunk_at(p, 0)).astype(jnp.bfloat16)
            rd[p][0] = mk(p, 0)
            rd[p][0].start()

        for s in range(1, N_DEV - 1):
            slot = s % 2
            parts = {}
            for p in order:
                parts[p] = part(p, chunk_at(p, s))
            for p in order:
                rd[p][s - 1].wait_recv()
                v = (parts[p] + rbs[p][1 - slot].astype(jnp.float32)
                     ).astype(jnp.bfloat16)
                if s >= 2:
                    rd[p][s - 2].wait_send()
                sbs[p][slot] = v
                if s <= N_DEV - 3:
                    pl.semaphore_signal(
                        creds[p], inc=1, device_id=(src(p),),
                        device_id_type=pl.DeviceIdType.MESH,
                    )
                if s >= 2:
                    pl.semaphore_wait(creds[p], 1)
                rd[p][s] = mk(p, slot)
                rd[p][s].start()

        me = lax.axis_index("i")
        last = (N_DEV - 2) % 2
        scale = sx_ref[0] * sw_ref[0]
        for p in order:
            pown = part(p, me)
            rd[p][N_DEV - 2].wait_recv()
            acc = pown + rbs[p][last].astype(jnp.float32)
            y = acc * scale
            out_ref[:, cols[p]:cols[p] + nq] = y * jax.nn.sigmoid(y)
        for p in range(N_PIPE):
            rd[p][N_DEV - 3].wait_send()
            rd[p][N_DEV - 2].wait_send()

    return pl.pallas_call(
        body,
        out_shape=jax.ShapeDtypeStruct((m_per, n), jnp.float32),
        in_specs=[
            pl.BlockSpec(memory_space=pltpu.VMEM),
            pl.BlockSpec(memory_space=pltpu.VMEM),
            pl.BlockSpec(memory_space=pltpu.SMEM),
            pl.BlockSpec(memory_space=pltpu.SMEM),
            pl.BlockSpec(memory_space=pltpu.SMEM),
        ],
        out_specs=pl.BlockSpec(memory_space=pltpu.VMEM),
        scratch_shapes=(
            [pltpu.VMEM((k_loc, n), jnp.bfloat16)]
            + [pltpu.VMEM((2, m_per, nq), jnp.bfloat16)] * N_PIPE
            + [pltpu.VMEM((2, m_per, nq), jnp.bfloat16)] * N_PIPE
            + [pltpu.SemaphoreType.DMA((2,))] * N_PIPE
            + [pltpu.SemaphoreType.DMA((2,))] * N_PIPE
            + [pltpu.SemaphoreType.REGULAR] * N_PIPE
        ),
        compiler_params=pltpu.CompilerParams(collective_id=0),
    )(x, w_mat, scale_x, scale_w, meta)
